# baseline (speedup 1.0000x reference)
import jax
import jax.numpy as jnp
import numpy as np
from jax.experimental import pallas as pl
from jax.experimental.pallas import tpu as pltpu

_N_PRED = 1000
_N_GT = 100
_GT_PAD = 128
_K = 128 * 128
_KT = 8
_KB = _KT * 128
_KSTEPS = 128 // _KT
_THRESHOLDS = [float(t) for t in np.linspace(0.5, 0.95, 10)]


def _map_kernel(p_ref, g_ref, out_ref, acc_ref, ag_ref):
    k = pl.program_id(0)

    @pl.when(k == 0)
    def _init():
        acc_ref[...] = jnp.zeros_like(acc_ref)
        ag_ref[...] = jnp.zeros_like(ag_ref)

    blk = (p_ref[...] > 0.5).astype(jnp.int8)
    pbin = blk.reshape(_N_PRED, _KB)
    gblk = (g_ref[...] > 0.5).astype(jnp.int8)
    graw = gblk.reshape(_N_GT, _KB)
    gbin = jnp.concatenate(
        [graw,
         jnp.zeros((_GT_PAD - _N_GT - 1, _KB), jnp.int8),
         jnp.ones((1, _KB), jnp.int8)], axis=0)
    acc_ref[...] += jax.lax.dot_general(
        pbin, gbin, (((1,), (1,)), ((), ())),
        preferred_element_type=jnp.int32)
    ones = jnp.ones((8, _KB), jnp.int8)
    ag_ref[...] += jax.lax.dot_general(
        ones, gbin, (((1,), (1,)), ((), ())),
        preferred_element_type=jnp.int32)

    @pl.when(k == _KSTEPS - 1)
    def _finalize():
        inter = acc_ref[...].astype(jnp.float32)
        area_p = inter[:, _GT_PAD - 1:_GT_PAD]
        area_g = ag_ref[0:1, :].astype(jnp.float32)
        union = area_p + area_g - inter
        iou = inter / jnp.maximum(union, 1e-9)
        col = jax.lax.broadcasted_iota(jnp.int32, (1, _GT_PAD), 1)
        colmask = (col < _N_GT).astype(jnp.float32)
        precs = []
        for t in _THRESHOLDS:
            mf = jnp.where(iou > t, 1.0, 0.0) * colmask
            tp = jnp.sum(jnp.max(mf, axis=0, keepdims=True))
            matched_pred = jnp.sum(jnp.max(mf, axis=1, keepdims=True))
            fp = float(_N_PRED) - matched_pred
            fn = float(_N_GT) - tp
            precs.append(tp / jnp.maximum(tp + fp + fn, 1e-9))
        map50 = precs[0]
        map50_95 = sum(precs) / float(len(precs))
        row = jnp.where(col == 0, map50, jnp.where(col == 1, map50_95, 0.0))
        out_ref[...] = jnp.broadcast_to(row, (8, _GT_PAD))


def kernel(predicted_masks, ground_truth_masks):
    P4 = predicted_masks.reshape(_N_PRED // 8, 8, 128, 128)
    out = pl.pallas_call(
        _map_kernel,
        grid=(_KSTEPS,),
        in_specs=[
            pl.BlockSpec((_N_PRED // 8, 8, _KT, 128), lambda k: (0, 0, k, 0)),
            pl.BlockSpec((_N_GT, _KT, 128), lambda k: (0, k, 0)),
        ],
        out_specs=pl.BlockSpec((8, _GT_PAD), lambda k: (0, 0)),
        out_shape=jax.ShapeDtypeStruct((8, _GT_PAD), jnp.float32),
        scratch_shapes=[
            pltpu.VMEM((_N_PRED, _GT_PAD), jnp.int32),
            pltpu.VMEM((8, _GT_PAD), jnp.int32),
        ],
    )(P4, ground_truth_masks)
    return (out[0, 0], out[0, 1])


# speedup vs baseline: 1.2627x; 1.2627x over previous
import jax
import jax.numpy as jnp
import numpy as np
from jax.experimental import pallas as pl
from jax.experimental.pallas import tpu as pltpu

_N_PRED = 1000
_N_GT = 100
_GT_PAD = 128
_K = 128 * 128
_KT = 32
_KB = _KT * 128
_KSTEPS = 128 // _KT
_THRESHOLDS = [float(t) for t in np.linspace(0.5, 0.95, 10)]


def _map_kernel(p_ref, g_ref, out_ref, acc_ref, ag_ref):
    k = pl.program_id(0)

    @pl.when(k == 0)
    def _init():
        acc_ref[...] = jnp.zeros_like(acc_ref)
        ag_ref[...] = jnp.zeros_like(ag_ref)

    blk = (p_ref[...] > 0.5).astype(jnp.int8)
    pbin = blk.reshape(_N_PRED, _KB)
    gblk = (g_ref[...] > 0.5).astype(jnp.int8)
    graw = gblk.reshape(_N_GT, _KB)
    gbin = jnp.concatenate(
        [graw,
         jnp.zeros((_GT_PAD - _N_GT - 1, _KB), jnp.int8),
         jnp.ones((1, _KB), jnp.int8)], axis=0)
    acc_ref[...] += jax.lax.dot_general(
        pbin, gbin, (((1,), (1,)), ((), ())),
        preferred_element_type=jnp.int32)
    ones = jnp.ones((8, _KB), jnp.int8)
    ag_ref[...] += jax.lax.dot_general(
        ones, gbin, (((1,), (1,)), ((), ())),
        preferred_element_type=jnp.int32)

    @pl.when(k == _KSTEPS - 1)
    def _finalize():
        inter = acc_ref[...].astype(jnp.float32)
        area_p = inter[:, _GT_PAD - 1:_GT_PAD]
        area_g = ag_ref[0:1, :].astype(jnp.float32)
        union = area_p + area_g - inter
        iou = inter / jnp.maximum(union, 1e-9)
        col = jax.lax.broadcasted_iota(jnp.int32, (1, _GT_PAD), 1)
        colmask = (col < _N_GT).astype(jnp.float32)
        precs = []
        for t in _THRESHOLDS:
            mf = jnp.where(iou > t, 1.0, 0.0) * colmask
            tp = jnp.sum(jnp.max(mf, axis=0, keepdims=True))
            matched_pred = jnp.sum(jnp.max(mf, axis=1, keepdims=True))
            fp = float(_N_PRED) - matched_pred
            fn = float(_N_GT) - tp
            precs.append(tp / jnp.maximum(tp + fp + fn, 1e-9))
        map50 = precs[0]
        map50_95 = sum(precs) / float(len(precs))
        row = jnp.where(col == 0, map50, jnp.where(col == 1, map50_95, 0.0))
        out_ref[...] = jnp.broadcast_to(row, (8, _GT_PAD))


def kernel(predicted_masks, ground_truth_masks):
    P4 = predicted_masks.reshape(_N_PRED // 8, 8, 128, 128)
    out = pl.pallas_call(
        _map_kernel,
        grid=(_KSTEPS,),
        in_specs=[
            pl.BlockSpec((_N_PRED // 8, 8, _KT, 128), lambda k: (0, 0, k, 0)),
            pl.BlockSpec((_N_GT, _KT, 128), lambda k: (0, k, 0)),
        ],
        out_specs=pl.BlockSpec((8, _GT_PAD), lambda k: (0, 0)),
        out_shape=jax.ShapeDtypeStruct((8, _GT_PAD), jnp.float32),
        scratch_shapes=[
            pltpu.VMEM((_N_PRED, _GT_PAD), jnp.int32),
            pltpu.VMEM((8, _GT_PAD), jnp.int32),
        ],
    )(P4, ground_truth_masks)
    return (out[0, 0], out[0, 1])
